# Initial kernel scaffold; baseline (speedup 1.0000x reference)
import jax
import jax.numpy as jnp
from jax.experimental import pallas as pl


def kernel(input_ids, pos_i, pos_j, emb_table, W, b):
    raise NotImplementedError("write your pallas kernel here")


# speedup vs baseline: 2.1068x; 2.1068x over previous
import functools

import jax
import jax.numpy as jnp
from jax import lax
from jax.experimental import pallas as pl
from jax.experimental.pallas import tpu as pltpu
from jax.experimental.pallas import tpu_sc as plsc

H = 64
LANES = 16
NC, NS = 2, 16
NW = NC * NS
CHUNK = 1024


def _proj_table_kernel(emb_ref, w1_ref, b_ref, p_ref):
    p_ref[...] = (
        lax.dot_general(
            emb_ref[...], w1_ref[...],
            (((1,), (1,)), ((), ())),
            preferred_element_type=jnp.float32,
        )
        + b_ref[...]
    )


def _sc_body(p_hbm, wij_hbm, ids_hbm, pi_hbm, pj_hbm, out_hbm,
             p_v, wij_v, ids_v, pi_v, pj_v, rows_v, *, tpw):
    wid = lax.axis_index("s") * NC + lax.axis_index("c")
    base = wid * tpw

    pltpu.sync_copy(p_hbm, p_v)
    pltpu.sync_copy(wij_hbm, wij_v)

    wi = [wij_v[0, pl.ds(fc * LANES, LANES)] for fc in range(H // LANES)]
    wj = [wij_v[1, pl.ds(fc * LANES, LANES)] for fc in range(H // LANES)]
    cols = [lax.iota(jnp.int32, LANES) + fc * LANES for fc in range(H // LANES)]

    def chunk_body(ci, _):
        tb = base + ci * CHUNK
        pltpu.sync_copy(ids_hbm.at[pl.ds(tb, CHUNK)], ids_v)
        pltpu.sync_copy(pi_hbm.at[pl.ds(tb, CHUNK)], pi_v)
        pltpu.sync_copy(pj_hbm.at[pl.ds(tb, CHUNK)], pj_v)

        def tok_body(t, _):
            tvec = jnp.full((LANES,), t, jnp.int32)
            idsp = plsc.load_gather(ids_v, [tvec])
            pisp = plsc.load_gather(pi_v, [tvec])
            pjsp = plsc.load_gather(pj_v, [tvec])
            for fc in range(H // LANES):
                g = plsc.load_gather(p_v, [idsp, cols[fc]])
                rows_v[pl.ds(t * H + fc * LANES, LANES)] = (
                    g + pisp * wi[fc] + pjsp * wj[fc]
                )
            return 0

        lax.fori_loop(0, CHUNK, tok_body, 0, unroll=4)
        pltpu.sync_copy(rows_v, out_hbm.at[pl.ds(tb * H, CHUNK * H)])
        return 0

    lax.fori_loop(0, tpw // CHUNK, chunk_body, 0)


def kernel(input_ids, pos_i, pos_j, emb_table, W, b):
    B, L = input_ids.shape
    N = B * L
    assert N % (NW * CHUNK) == 0
    tpw = N // NW

    ids = input_ids.reshape(-1).astype(jnp.int32)
    pi = pos_i.reshape(-1)
    pj = pos_j.reshape(-1)

    vocab = emb_table.shape[0]
    emb_pad = jnp.pad(emb_table, ((0, 16 - vocab), (0, 0)))
    w1 = W[:, :H]
    wij = W[:, H:H + 2].T
    p_tab = pl.pallas_call(
        _proj_table_kernel,
        out_shape=jax.ShapeDtypeStruct((16, H), jnp.float32),
    )(emb_pad, w1, b.reshape(1, H))

    mesh = plsc.VectorSubcoreMesh(
        core_axis_name="c", subcore_axis_name="s",
        num_cores=NC, num_subcores=NS,
    )
    sc = pl.kernel(
        functools.partial(_sc_body, tpw=tpw),
        out_type=jax.ShapeDtypeStruct((N * H,), jnp.float32),
        mesh=mesh,
        scratch_types=[
            pltpu.VMEM((16, H), jnp.float32),
            pltpu.VMEM((2, H), jnp.float32),
            pltpu.VMEM((CHUNK,), jnp.int32),
            pltpu.VMEM((CHUNK,), jnp.float32),
            pltpu.VMEM((CHUNK,), jnp.float32),
            pltpu.VMEM((CHUNK * H,), jnp.float32),
        ],
        compiler_params=pltpu.CompilerParams(needs_layout_passes=False),
    )
    out_flat = sc(p_tab, wij, ids, pi, pj)
    return out_flat.reshape(B, L, H)
